# baseline (speedup 1.0000x reference)
import functools

import jax
import jax.numpy as jnp
from jax import lax
from jax.experimental import pallas as pl
from jax.experimental.pallas import tpu as pltpu
from jax.experimental.pallas import tpu_sc as plsc

D = 128
NC = 2
NS = 16
NW = NC * NS
EBLK = 40


def _pq_body(x_ref, wa_ref, wb_ref, b1_ref, p_ref, q_ref):
    xb = x_ref[...]
    p_ref[...] = jnp.dot(xb, wa_ref[...], preferred_element_type=jnp.float32)
    q_ref[...] = (
        jnp.dot(xb, wb_ref[...], preferred_element_type=jnp.float32) + b1_ref[...]
    )


def _pq(x, w1at, w1bt, b1row, blk):
    n = x.shape[0]
    return pl.pallas_call(
        _pq_body,
        grid=(n // blk,),
        in_specs=[
            pl.BlockSpec((blk, D), lambda i: (i, 0)),
            pl.BlockSpec((D, D), lambda i: (0, 0)),
            pl.BlockSpec((D, D), lambda i: (0, 0)),
            pl.BlockSpec((1, D), lambda i: (0, 0)),
        ],
        out_specs=[
            pl.BlockSpec((blk, D), lambda i: (i, 0)),
            pl.BlockSpec((blk, D), lambda i: (i, 0)),
        ],
        out_shape=[
            jax.ShapeDtypeStruct((n, D), jnp.float32),
            jax.ShapeDtypeStruct((n, D), jnp.float32),
        ],
    )(x, w1at, w1bt, b1row)


def _make_edge_kernel(n, e):
    epw = e // NW
    nblk = epw // EBLK
    rps = ((n // NS) + 7) // 8 * 8
    npad = rps * NS
    drows = (npad // D + 7) // 8 * 8
    mesh = plsc.VectorSubcoreMesh(core_axis_name="c", subcore_axis_name="s")

    ebuf = lambda: pltpu.VMEM((EBLK, D), jnp.float32)
    ibuf = lambda: pltpu.VMEM((EBLK + 16,), jnp.int32)

    @functools.partial(
        pl.kernel,
        out_type=[
            jax.ShapeDtypeStruct((NC, npad, D), jnp.float32),
            jax.ShapeDtypeStruct((NW, drows * D), jnp.float32),
        ],
        mesh=mesh,
        compiler_params=pltpu.CompilerParams(needs_layout_passes=False),
        scratch_types=(
            [ibuf(), ibuf(), pltpu.VMEM((EBLK + 16,), jnp.float32)] * 3
            + [ebuf(), ebuf()] * 3
            + [ebuf(), pltpu.VMEM((EBLK + 16,), jnp.int32)]
            + [pltpu.VMEM((drows * D,), jnp.float32),
               pltpu.VMEM_SHARED((npad, D), jnp.float32)]
            + [pltpu.SemaphoreType.DMA] * 16
        ),
    )
    def edge_kernel(p_hbm, q_hbm, row_hbm, col_hbm, attr_hbm, zero_hbm, zerod_hbm,
                    outh_hbm, outd_hbm, *sc):
        IX = [sc[3 * u:3 * u + 3] + sc[19 + 3 * u:19 + 3 * u + 3] for u in range(3)]
        PQ = [sc[9 + 2 * u:9 + 2 * u + 2] + sc[28 + 2 * u:28 + 2 * u + 2]
              for u in range(3)]
        hbuf, cs = sc[15], sc[16]
        degl, hacc = sc[17], sc[18]
        sem_s = sc[34]

        cid = lax.axis_index("c")
        sid = lax.axis_index("s")
        wid = cid * NS + sid

        rbase = sid * rps
        pltpu.sync_copy(zero_hbm.at[pl.ds(rbase, rps)], hacc.at[pl.ds(rbase, rps)])
        pltpu.sync_copy(zerod_hbm, degl)
        plsc.subcore_barrier()

        ones = jnp.ones((16,), jnp.float32)
        lane0 = lax.iota(jnp.int32, 16) == 0
        ebase = wid * epw

        ids = pl.ds(0, EBLK)

        def fire_idx(b, I):
            off = ebase + b * EBLK
            pltpu.async_copy(row_hbm.at[pl.ds(off, EBLK)], I[0].at[ids], I[3])
            pltpu.async_copy(col_hbm.at[pl.ds(off, EBLK)], I[1].at[ids], I[4])
            pltpu.async_copy(attr_hbm.at[pl.ds(off, EBLK)], I[2].at[ids], I[5])

        def wait_idx(I):
            pltpu.make_async_copy(row_hbm.at[pl.ds(0, EBLK)], I[0].at[ids], I[3]).wait()
            pltpu.make_async_copy(col_hbm.at[pl.ds(0, EBLK)], I[1].at[ids], I[4]).wait()
            pltpu.make_async_copy(attr_hbm.at[pl.ds(0, EBLK)], I[2].at[ids], I[5]).wait()

        def fire_gather(I, G):
            pltpu.async_copy(p_hbm.at[I[0].at[ids]], G[0], G[2])
            pltpu.async_copy(q_hbm.at[I[1].at[ids]], G[1], G[3])

        def wait_gather(G):
            pltpu.make_async_copy(p_hbm.at[IX[0][0].at[ids]], G[0], G[2]).wait()
            pltpu.make_async_copy(q_hbm.at[IX[0][1].at[ids]], G[1], G[3]).wait()

        def fire_scatter():
            pltpu.async_copy(hbuf, hacc.at[cs.at[ids]], sem_s, add=True)

        def wait_scatter():
            pltpu.make_async_copy(hbuf, hacc.at[cs.at[ids]], sem_s).wait()

        def compute(I, G):
            cidx, attr_v = I[1], I[2]
            pbuf, qbuf = G[0], G[1]
            for j in range((EBLK + 15) // 16):
                a16 = attr_v[pl.ds(j * 16, 16)]
                c16 = cidx[pl.ds(j * 16, 16)]
                for k in range(min(16, EBLK - j * 16)):
                    i = j * 16 + k
                    av = jnp.full((16,), a16[k], dtype=jnp.float32)
                    for g in range(D // 16):
                        sl = pl.ds(g * 16, 16)
                        hbuf[i, sl] = jnp.maximum(
                            pbuf[i, sl] * av + qbuf[i, sl], 0.0)
                    cvec = jnp.full((16,), c16[k], dtype=jnp.int32)
                    plsc.addupdate_scatter(degl, [cvec], ones, mask=lane0)

        def process(b, u):
            u2 = (u + 2) % 3

            @pl.when(b + 2 < nblk)
            def _():
                fire_idx(b + 2, IX[u2])

            wait_gather(PQ[u])

            @pl.when(b >= 1)
            def _():
                wait_scatter()

            for j in range((EBLK + 15) // 16):
                cs[pl.ds(j * 16, 16)] = IX[u][1][pl.ds(j * 16, 16)]

            compute(IX[u], PQ[u])
            fire_scatter()

            @pl.when(b + 2 < nblk)
            def _():
                wait_idx(IX[u2])
                fire_gather(IX[u2], PQ[u2])

        fire_idx(0, IX[0])
        fire_idx(1, IX[1])
        wait_idx(IX[0])
        fire_gather(IX[0], PQ[0])
        wait_idx(IX[1])
        fire_gather(IX[1], PQ[1])

        def outer(t, _):
            b3 = 3 * t
            for u in range(3):
                @pl.when(b3 + u < nblk)
                def _(u=u):
                    process(b3 + u, u)

            return 0

        lax.fori_loop(0, (nblk + 2) // 3, outer, 0)

        wait_scatter()
        plsc.subcore_barrier()

        pltpu.sync_copy(hacc.at[pl.ds(rbase, rps)],
                        outh_hbm.at[cid, pl.ds(rbase, rps)])
        pltpu.sync_copy(degl, outd_hbm.at[wid])

    return edge_kernel


def _out_body(h0_ref, h1_ref, d_ref, w_ref, b2_ref, o_ref):
    deg = jnp.sum(d_ref[...], axis=1, keepdims=True)
    o_ref[...] = (
        jnp.dot(h0_ref[...] + h1_ref[...], w_ref[...],
                preferred_element_type=jnp.float32)
        + deg * b2_ref[...]
    )


def _final(h0, h1, degt, w2t, b2row, n, blk):
    return pl.pallas_call(
        _out_body,
        grid=(n // blk,),
        in_specs=[
            pl.BlockSpec((blk, D), lambda i: (i, 0)),
            pl.BlockSpec((blk, D), lambda i: (i, 0)),
            pl.BlockSpec((blk, NW), lambda i: (i, 0)),
            pl.BlockSpec((D, D), lambda i: (0, 0)),
            pl.BlockSpec((1, D), lambda i: (0, 0)),
        ],
        out_specs=pl.BlockSpec((blk, D), lambda i: (i, 0)),
        out_shape=jax.ShapeDtypeStruct((n, D), jnp.float32),
    )(h0, h1, degt, w2t, b2row)


def kernel(x, edge_index, edge_attr, W1, b1, W2, b2):
    n = x.shape[0]
    e = edge_index.shape[1]
    rps = ((n // NS) + 7) // 8 * 8
    npad = rps * NS
    drows = (npad // D + 7) // 8 * 8

    row = edge_index[0].astype(jnp.int32)
    col = edge_index[1].astype(jnp.int32)
    w1at = W1[:, :D].T
    w1bt = W1[:, D:].T
    b1row = b1[None, :]
    w2t = W2.T
    b2row = b2[None, :]
    zeros = jnp.zeros((npad, D), jnp.float32)
    zerod = jnp.zeros((drows * D,), jnp.float32)

    p, q = _pq(x, w1at, w1bt, b1row, blk=2000)
    h, deg = _make_edge_kernel(n, e)(p, q, row, col, edge_attr, zeros, zerod)
    degt = deg[:, :n].T
    return _final(h[0], h[1], degt, w2t, b2row, n, blk=2000)


# speedup vs baseline: 1.0528x; 1.0528x over previous
import functools

import jax
import jax.numpy as jnp
from jax import lax
from jax.experimental import pallas as pl
from jax.experimental.pallas import tpu as pltpu
from jax.experimental.pallas import tpu_sc as plsc

D = 128
NC = 2
NS = 16
NW = NC * NS
EBLK = 40


def _pq_body(x_ref, wa_ref, wb_ref, b1_ref, p_ref, q_ref):
    xb = x_ref[...]
    p_ref[...] = jnp.dot(xb, wa_ref[...], preferred_element_type=jnp.float32)
    q_ref[...] = (
        jnp.dot(xb, wb_ref[...], preferred_element_type=jnp.float32) + b1_ref[...]
    )


def _pq(x, w1at, w1bt, b1row, blk):
    n = x.shape[0]
    return pl.pallas_call(
        _pq_body,
        grid=(n // blk,),
        in_specs=[
            pl.BlockSpec((blk, D), lambda i: (i, 0)),
            pl.BlockSpec((D, D), lambda i: (0, 0)),
            pl.BlockSpec((D, D), lambda i: (0, 0)),
            pl.BlockSpec((1, D), lambda i: (0, 0)),
        ],
        out_specs=[
            pl.BlockSpec((blk, D), lambda i: (i, 0)),
            pl.BlockSpec((blk, D), lambda i: (i, 0)),
        ],
        out_shape=[
            jax.ShapeDtypeStruct((n, D), jnp.float32),
            jax.ShapeDtypeStruct((n, D), jnp.float32),
        ],
    )(x, w1at, w1bt, b1row)


def _make_edge_kernel(n, e):
    epw = e // NW
    nblk = epw // EBLK
    rps = ((n // NS) + 7) // 8 * 8
    npad = rps * NS
    drows = (npad // D + 7) // 8 * 8
    mesh = plsc.VectorSubcoreMesh(core_axis_name="c", subcore_axis_name="s")

    ebuf = lambda: pltpu.VMEM((EBLK, D), jnp.int32)
    hbf = lambda: pltpu.VMEM((EBLK, D), jnp.float32)
    ibuf = lambda: pltpu.VMEM((EBLK + 16,), jnp.int32)

    @functools.partial(
        pl.kernel,
        out_type=[
            jax.ShapeDtypeStruct((NC, npad, D), jnp.float32),
            jax.ShapeDtypeStruct((NW, drows * D), jnp.float32),
        ],
        mesh=mesh,
        compiler_params=pltpu.CompilerParams(needs_layout_passes=False),
        scratch_types=(
            [ibuf(), ibuf(), pltpu.VMEM((EBLK + 16,), jnp.float32)] * 3
            + [ebuf(), ebuf()] * 3
            + [hbf(), pltpu.VMEM((EBLK + 16,), jnp.int32)]
            + [pltpu.VMEM((drows * D,), jnp.float32),
               pltpu.VMEM_SHARED((npad, D), jnp.float32)]
            + [pltpu.SemaphoreType.DMA] * 16
        ),
    )
    def edge_kernel(t_hbm, row_hbm, col_hbm, attr_hbm, zero_hbm, zerod_hbm,
                    outh_hbm, outd_hbm, *sc):
        IX = [sc[3 * u:3 * u + 3] + sc[19 + 3 * u:19 + 3 * u + 3] for u in range(3)]
        PQ = [sc[9 + 2 * u:9 + 2 * u + 2] + sc[28 + 2 * u:28 + 2 * u + 2]
              for u in range(3)]
        hbuf, cs = sc[15], sc[16]
        degl, hacc = sc[17], sc[18]
        sem_s = sc[34]

        cid = lax.axis_index("c")
        sid = lax.axis_index("s")
        wid = cid * NS + sid

        rbase = sid * rps
        pltpu.sync_copy(zero_hbm.at[pl.ds(rbase, rps)], hacc.at[pl.ds(rbase, rps)])
        pltpu.sync_copy(zerod_hbm, degl)
        plsc.subcore_barrier()

        ones = jnp.ones((16,), jnp.float32)
        lane0 = lax.iota(jnp.int32, 16) == 0
        ebase = wid * epw

        ids = pl.ds(0, EBLK)

        def fire_idx(b, I):
            off = ebase + b * EBLK
            pltpu.async_copy(row_hbm.at[pl.ds(off, EBLK)], I[0].at[ids], I[3])
            pltpu.async_copy(col_hbm.at[pl.ds(off, EBLK)], I[1].at[ids], I[4])
            pltpu.async_copy(attr_hbm.at[pl.ds(off, EBLK)], I[2].at[ids], I[5])

        def wait_idx(I):
            pltpu.make_async_copy(row_hbm.at[pl.ds(0, EBLK)], I[0].at[ids], I[3]).wait()
            pltpu.make_async_copy(col_hbm.at[pl.ds(0, EBLK)], I[1].at[ids], I[4]).wait()
            pltpu.make_async_copy(attr_hbm.at[pl.ds(0, EBLK)], I[2].at[ids], I[5]).wait()

        def fire_gather(I, G):
            pltpu.async_copy(t_hbm.at[I[0].at[ids]], G[0], G[2])
            pltpu.async_copy(t_hbm.at[I[1].at[ids]], G[1], G[3])

        def wait_gather(G):
            pltpu.make_async_copy(t_hbm.at[IX[0][0].at[ids]], G[0], G[2]).wait()
            pltpu.make_async_copy(t_hbm.at[IX[0][1].at[ids]], G[1], G[3]).wait()

        def fire_scatter():
            pltpu.async_copy(hbuf, hacc.at[cs.at[ids]], sem_s, add=True)

        def wait_scatter():
            pltpu.make_async_copy(hbuf, hacc.at[cs.at[ids]], sem_s).wait()

        def compute(I, G):
            cidx, attr_v = I[1], I[2]
            tr, tc = G[0], G[1]
            for j in range((EBLK + 15) // 16):
                a16 = attr_v[pl.ds(j * 16, 16)]
                c16 = cidx[pl.ds(j * 16, 16)]
                for k in range(min(16, EBLK - j * 16)):
                    i = j * 16 + k
                    av16 = jnp.full((16,), a16[k], dtype=jnp.float32)
                    av = plsc.pack(av16, av16,
                                   format=plsc.PackFormat.INTERLEAVED)
                    for g in range(D // 32):
                        pb = plsc.bitcast(tr[i, pl.ds(g * 16, 16)],
                                          jnp.bfloat16)
                        qb = plsc.bitcast(tc[i, pl.ds(D // 2 + g * 16, 16)],
                                          jnp.bfloat16)
                        hb = jnp.maximum(pb * av + qb, jnp.bfloat16(0.0))
                        lo, hi = plsc.unpack(
                            hb, format=plsc.PackFormat.INTERLEAVED)
                        hbuf[i, pl.ds(g * 32, 16)] = lo
                        hbuf[i, pl.ds(g * 32 + 16, 16)] = hi
                    cvec = jnp.full((16,), c16[k], dtype=jnp.int32)
                    plsc.addupdate_scatter(degl, [cvec], ones, mask=lane0)

        def process(b, u):
            u2 = (u + 2) % 3

            @pl.when(b + 2 < nblk)
            def _():
                fire_idx(b + 2, IX[u2])

            wait_gather(PQ[u])

            @pl.when(b >= 1)
            def _():
                wait_scatter()

            for j in range((EBLK + 15) // 16):
                cs[pl.ds(j * 16, 16)] = IX[u][1][pl.ds(j * 16, 16)]

            compute(IX[u], PQ[u])
            fire_scatter()

            @pl.when(b + 2 < nblk)
            def _():
                wait_idx(IX[u2])
                fire_gather(IX[u2], PQ[u2])

        fire_idx(0, IX[0])
        fire_idx(1, IX[1])
        wait_idx(IX[0])
        fire_gather(IX[0], PQ[0])
        wait_idx(IX[1])
        fire_gather(IX[1], PQ[1])

        def outer(t, _):
            b3 = 3 * t
            for u in range(3):
                @pl.when(b3 + u < nblk)
                def _(u=u):
                    process(b3 + u, u)

            return 0

        lax.fori_loop(0, (nblk + 2) // 3, outer, 0)

        wait_scatter()
        plsc.subcore_barrier()

        pltpu.sync_copy(hacc.at[pl.ds(rbase, rps)],
                        outh_hbm.at[cid, pl.ds(rbase, rps)])
        pltpu.sync_copy(degl, outd_hbm.at[wid])

    return edge_kernel


def _out_body(h0_ref, h1_ref, d_ref, w_ref, b2_ref, o_ref):
    deg = jnp.sum(d_ref[...], axis=1, keepdims=True)
    o_ref[...] = (
        jnp.dot(h0_ref[...] + h1_ref[...], w_ref[...],
                preferred_element_type=jnp.float32)
        + deg * b2_ref[...]
    )


def _final(h0, h1, degt, w2t, b2row, n, blk):
    return pl.pallas_call(
        _out_body,
        grid=(n // blk,),
        in_specs=[
            pl.BlockSpec((blk, D), lambda i: (i, 0)),
            pl.BlockSpec((blk, D), lambda i: (i, 0)),
            pl.BlockSpec((blk, NW), lambda i: (i, 0)),
            pl.BlockSpec((D, D), lambda i: (0, 0)),
            pl.BlockSpec((1, D), lambda i: (0, 0)),
        ],
        out_specs=pl.BlockSpec((blk, D), lambda i: (i, 0)),
        out_shape=jax.ShapeDtypeStruct((n, D), jnp.float32),
    )(h0, h1, degt, w2t, b2row)


def kernel(x, edge_index, edge_attr, W1, b1, W2, b2):
    n = x.shape[0]
    e = edge_index.shape[1]
    rps = ((n // NS) + 7) // 8 * 8
    npad = rps * NS
    drows = (npad // D + 7) // 8 * 8

    row = edge_index[0].astype(jnp.int32)
    col = edge_index[1].astype(jnp.int32)
    w1at = W1[:, :D].T
    w1bt = W1[:, D:].T
    b1row = b1[None, :]
    perm = jnp.arange(D)
    grp, off = perm // 32, perm % 32
    stored_feat = grp * 32 + jnp.where(off < 16, 2 * off, 2 * (off - 16) + 1)
    w2t = W2.T[stored_feat]
    b2row = b2[None, :]
    zeros = jnp.zeros((npad, D), jnp.float32)
    zerod = jnp.zeros((drows * D,), jnp.float32)

    p, q = _pq(x, w1at, w1bt, b1row, blk=2000)
    pw = lax.bitcast_convert_type(
        p.astype(jnp.bfloat16).reshape(n, D // 2, 2), jnp.int32)
    qw = lax.bitcast_convert_type(
        q.astype(jnp.bfloat16).reshape(n, D // 2, 2), jnp.int32)
    t = jnp.concatenate([pw, qw], axis=1)
    h, deg = _make_edge_kernel(n, e)(t, row, col, edge_attr, zeros, zerod)
    degt = deg[:, :n].T
    return _final(h[0], h[1], degt, w2t, b2row, n, blk=2000)


# speedup vs baseline: 1.5298x; 1.4532x over previous
import functools

import jax
import jax.numpy as jnp
from jax import lax
from jax.experimental import pallas as pl
from jax.experimental.pallas import tpu as pltpu
from jax.experimental.pallas import tpu_sc as plsc

D = 128
NC = 2
NS = 16
NW = NC * NS
EBLK = 40


def _pq_body(x_ref, wa_ref, wb_ref, b1_ref, p_ref, q_ref):
    xb = x_ref[...]
    p_ref[...] = jnp.dot(xb, wa_ref[...], preferred_element_type=jnp.float32)
    q_ref[...] = (
        jnp.dot(xb, wb_ref[...], preferred_element_type=jnp.float32) + b1_ref[...]
    )


def _pq(x, w1at, w1bt, b1row, blk):
    n = x.shape[0]
    return pl.pallas_call(
        _pq_body,
        grid=(n // blk,),
        in_specs=[
            pl.BlockSpec((blk, D), lambda i: (i, 0)),
            pl.BlockSpec((D, D), lambda i: (0, 0)),
            pl.BlockSpec((D, D), lambda i: (0, 0)),
            pl.BlockSpec((1, D), lambda i: (0, 0)),
        ],
        out_specs=[
            pl.BlockSpec((blk, D), lambda i: (i, 0)),
            pl.BlockSpec((blk, D), lambda i: (i, 0)),
        ],
        out_shape=[
            jax.ShapeDtypeStruct((n, D), jnp.float32),
            jax.ShapeDtypeStruct((n, D), jnp.float32),
        ],
    )(x, w1at, w1bt, b1row)


def _make_edge_kernel(n, e):
    epw = e // NW
    nblk = epw // EBLK
    rps = ((n // NS) + 7) // 8 * 8
    npad = rps * NS
    drows = (npad // D + 7) // 8 * 8
    mesh = plsc.VectorSubcoreMesh(core_axis_name="c", subcore_axis_name="s")

    ebuf = lambda: pltpu.VMEM((EBLK, D), jnp.int32)
    hbf = lambda: pltpu.VMEM((EBLK, D), jnp.float32)
    ibuf = lambda: pltpu.VMEM((EBLK + 16,), jnp.int32)

    @functools.partial(
        pl.kernel,
        out_type=[
            jax.ShapeDtypeStruct((NC, npad, D), jnp.float32),
            jax.ShapeDtypeStruct((NW, drows * D), jnp.float32),
        ],
        mesh=mesh,
        compiler_params=pltpu.CompilerParams(needs_layout_passes=False),
        scratch_types=(
            [ibuf(), ibuf(), pltpu.VMEM((EBLK + 16,), jnp.float32)] * 3
            + [ebuf(), ebuf()] * 3
            + [hbf(), pltpu.VMEM((EBLK + 16,), jnp.int32)]
            + [pltpu.VMEM((drows * D,), jnp.float32),
               pltpu.VMEM_SHARED((npad, D), jnp.float32)]
            + [pltpu.SemaphoreType.DMA] * 16
        ),
    )
    def edge_kernel(t_hbm, row_hbm, col_hbm, attr_hbm, zero_hbm, zerod_hbm,
                    outh_hbm, outd_hbm, *sc):
        IX = [sc[3 * u:3 * u + 3] + sc[19 + 3 * u:19 + 3 * u + 3] for u in range(3)]
        PQ = [sc[9 + 2 * u:9 + 2 * u + 2] + sc[28 + 2 * u:28 + 2 * u + 2]
              for u in range(3)]
        hbuf, cs = sc[15], sc[16]
        degl, hacc = sc[17], sc[18]
        sem_s = sc[34]

        cid = lax.axis_index("c")
        sid = lax.axis_index("s")
        wid = cid * NS + sid

        rbase = sid * rps
        pltpu.sync_copy(zero_hbm.at[pl.ds(rbase, rps)], hacc.at[pl.ds(rbase, rps)])
        pltpu.sync_copy(zerod_hbm, degl)
        plsc.subcore_barrier()

        ones = jnp.ones((16,), jnp.float32)
        lane0 = lax.iota(jnp.int32, 16) == 0
        ebase = wid * epw

        ids = pl.ds(0, EBLK)

        def fire_idx(b, I):
            off = ebase + b * EBLK
            pltpu.async_copy(row_hbm.at[pl.ds(off, EBLK)], I[0].at[ids], I[3])
            pltpu.async_copy(col_hbm.at[pl.ds(off, EBLK)], I[1].at[ids], I[4])
            pltpu.async_copy(attr_hbm.at[pl.ds(off, EBLK)], I[2].at[ids], I[5])

        def wait_idx(I):
            pltpu.make_async_copy(row_hbm.at[pl.ds(0, EBLK)], I[0].at[ids], I[3]).wait()
            pltpu.make_async_copy(col_hbm.at[pl.ds(0, EBLK)], I[1].at[ids], I[4]).wait()
            pltpu.make_async_copy(attr_hbm.at[pl.ds(0, EBLK)], I[2].at[ids], I[5]).wait()

        def fire_gather(I, G):
            pltpu.async_copy(t_hbm.at[I[0].at[ids]], G[0], G[2])
            pltpu.async_copy(t_hbm.at[I[1].at[ids]], G[1], G[3])

        def wait_gather(G):
            pltpu.make_async_copy(t_hbm.at[IX[0][0].at[ids]], G[0], G[2]).wait()
            pltpu.make_async_copy(t_hbm.at[IX[0][1].at[ids]], G[1], G[3]).wait()

        def fire_scatter():
            pltpu.async_copy(hbuf, hacc.at[cs.at[ids]], sem_s, add=True)

        def wait_scatter():
            pltpu.make_async_copy(hbuf, hacc.at[cs.at[ids]], sem_s).wait()

        def compute(I, G):
            cidx, attr_v = I[1], I[2]
            tr, tc = G[0], G[1]
            for j in range((EBLK + 15) // 16):
                a16 = attr_v[pl.ds(j * 16, 16)]
                c16 = cidx[pl.ds(j * 16, 16)]
                nk = min(16, EBLK - j * 16)
                kmask = lax.iota(jnp.int32, 16) < nk
                plsc.addupdate_scatter(degl, [c16], ones, mask=kmask)
                for k in range(nk):
                    i = j * 16 + k
                    av16 = jnp.full((16,), a16[k], dtype=jnp.float32)
                    av = plsc.pack(av16, av16,
                                   format=plsc.PackFormat.INTERLEAVED)
                    for g in range(D // 32):
                        pb = plsc.bitcast(tr[i, pl.ds(g * 16, 16)],
                                          jnp.bfloat16)
                        qb = plsc.bitcast(tc[i, pl.ds(D // 2 + g * 16, 16)],
                                          jnp.bfloat16)
                        hb = jnp.maximum(pb * av + qb, jnp.bfloat16(0.0))
                        lo, hi = plsc.unpack(
                            hb, format=plsc.PackFormat.INTERLEAVED)
                        hbuf[i, pl.ds(g * 32, 16)] = lo
                        hbuf[i, pl.ds(g * 32 + 16, 16)] = hi

        def process(b, u):
            u2 = (u + 2) % 3

            @pl.when(b + 2 < nblk)
            def _():
                fire_idx(b + 2, IX[u2])

            wait_gather(PQ[u])

            @pl.when(b >= 1)
            def _():
                wait_scatter()

            for j in range((EBLK + 15) // 16):
                cs[pl.ds(j * 16, 16)] = IX[u][1][pl.ds(j * 16, 16)]

            compute(IX[u], PQ[u])
            fire_scatter()

            @pl.when(b + 2 < nblk)
            def _():
                wait_idx(IX[u2])
                fire_gather(IX[u2], PQ[u2])

        fire_idx(0, IX[0])
        fire_idx(1, IX[1])
        wait_idx(IX[0])
        fire_gather(IX[0], PQ[0])
        wait_idx(IX[1])
        fire_gather(IX[1], PQ[1])

        def outer(t, _):
            b3 = 3 * t
            for u in range(3):
                @pl.when(b3 + u < nblk)
                def _(u=u):
                    process(b3 + u, u)

            return 0

        lax.fori_loop(0, (nblk + 2) // 3, outer, 0)

        wait_scatter()
        plsc.subcore_barrier()

        pltpu.sync_copy(hacc.at[pl.ds(rbase, rps)],
                        outh_hbm.at[cid, pl.ds(rbase, rps)])
        pltpu.sync_copy(degl, outd_hbm.at[wid])

    return edge_kernel


def _out_body(h0_ref, h1_ref, d_ref, w_ref, b2_ref, o_ref):
    deg = jnp.sum(d_ref[...], axis=1, keepdims=True)
    o_ref[...] = (
        jnp.dot(h0_ref[...] + h1_ref[...], w_ref[...],
                preferred_element_type=jnp.float32)
        + deg * b2_ref[...]
    )


def _final(h0, h1, degt, w2t, b2row, n, blk):
    return pl.pallas_call(
        _out_body,
        grid=(n // blk,),
        in_specs=[
            pl.BlockSpec((blk, D), lambda i: (i, 0)),
            pl.BlockSpec((blk, D), lambda i: (i, 0)),
            pl.BlockSpec((blk, NW), lambda i: (i, 0)),
            pl.BlockSpec((D, D), lambda i: (0, 0)),
            pl.BlockSpec((1, D), lambda i: (0, 0)),
        ],
        out_specs=pl.BlockSpec((blk, D), lambda i: (i, 0)),
        out_shape=jax.ShapeDtypeStruct((n, D), jnp.float32),
    )(h0, h1, degt, w2t, b2row)


def kernel(x, edge_index, edge_attr, W1, b1, W2, b2):
    n = x.shape[0]
    e = edge_index.shape[1]
    rps = ((n // NS) + 7) // 8 * 8
    npad = rps * NS
    drows = (npad // D + 7) // 8 * 8

    row = edge_index[0].astype(jnp.int32)
    col = edge_index[1].astype(jnp.int32)
    w1at = W1[:, :D].T
    w1bt = W1[:, D:].T
    b1row = b1[None, :]
    perm = jnp.arange(D)
    grp, off = perm // 32, perm % 32
    stored_feat = grp * 32 + jnp.where(off < 16, 2 * off, 2 * (off - 16) + 1)
    w2t = W2.T[stored_feat]
    b2row = b2[None, :]
    zeros = jnp.zeros((npad, D), jnp.float32)
    zerod = jnp.zeros((drows * D,), jnp.float32)

    p, q = _pq(x, w1at, w1bt, b1row, blk=2000)
    pw = lax.bitcast_convert_type(
        p.astype(jnp.bfloat16).reshape(n, D // 2, 2), jnp.int32)
    qw = lax.bitcast_convert_type(
        q.astype(jnp.bfloat16).reshape(n, D // 2, 2), jnp.int32)
    t = jnp.concatenate([pw, qw], axis=1)
    h, deg = _make_edge_kernel(n, e)(t, row, col, edge_attr, zeros, zerod)
    degt = deg[:, :n].T
    return _final(h[0], h[1], degt, w2t, b2row, n, blk=2000)
